# baseline (speedup 1.0000x reference)
import functools

import jax
import jax.numpy as jnp
import numpy as np
from jax import lax
from jax.experimental import pallas as pl
from jax.experimental.pallas import tpu as pltpu
from jax.experimental.pallas import tpu_sc as plsc

_MASK_PCT = 0.75
_B = 64
_NP = 1024
_D = 192
_N_MASKED = int(_MASK_PCT * _NP)
_N_UNMASKED = _NP - _N_MASKED
_NW = 32
_SC_B = 40
_TC_B = _B - _SC_B
_SC_ROWS = _SC_B * _D
_ROWS_PER_TILE = _SC_ROWS // _NW
_RCHUNK = 24
_NCHUNK = _ROWS_PER_TILE // _RCHUNK
_LANES = 16


def _threefry2x32(k0, k1, x0, x1):
    rot = ((13, 15, 26, 6), (17, 29, 16, 24))
    ks = [np.uint32(k0), np.uint32(k1),
          np.uint32(0x1BD11BDA) ^ np.uint32(k0) ^ np.uint32(k1)]
    x0 = (np.asarray(x0, np.uint32) + ks[0]).astype(np.uint32)
    x1 = (np.asarray(x1, np.uint32) + ks[1]).astype(np.uint32)
    for i in range(5):
        for r in rot[i % 2]:
            x0 = (x0 + x1).astype(np.uint32)
            x1 = ((x1 << np.uint32(r)) | (x1 >> np.uint32(32 - r))).astype(np.uint32)
            x1 = (x1 ^ x0).astype(np.uint32)
        x0 = (x0 + ks[(i + 1) % 3]).astype(np.uint32)
        x1 = (x1 + ks[(i + 2) % 3] + np.uint32(i + 1)).astype(np.uint32)
    return x0, x1


@functools.lru_cache(maxsize=1)
def _mask_constants():
    k0, k1 = _threefry2x32(0, 0, 0, 1)
    i = np.arange(_B * _NP, dtype=np.uint64)
    b0, b1 = _threefry2x32(k0, k1,
                           (i >> np.uint64(32)).astype(np.uint32),
                           (i & np.uint64(0xFFFFFFFF)).astype(np.uint32))
    bits = (b0 ^ b1).reshape(_B, _NP)
    scores = (((bits >> np.uint32(9)) | np.uint32(0x3F800000)).view(np.float32)
              - np.float32(1.0))
    perm = np.argsort(scores, axis=1, kind="stable")
    masked = np.sort(perm[:, :_N_MASKED], axis=1).astype(np.int32)
    unmasked = np.sort(perm[:, _N_MASKED:], axis=1).astype(np.int32)
    bool_mask = np.ones((_B, _NP), dtype=bool)
    np.put_along_axis(bool_mask, unmasked, False, axis=1)
    return masked, unmasked, bool_mask


_CONSTS = _mask_constants()

_sc_mesh = plsc.VectorSubcoreMesh(core_axis_name="c", subcore_axis_name="s")


@functools.partial(
    pl.kernel,
    mesh=_sc_mesh,
    compiler_params=pltpu.CompilerParams(needs_layout_passes=False),
    out_type=jax.ShapeDtypeStruct((_SC_ROWS, _N_UNMASKED), jnp.float32),
    scratch_types=[
        pltpu.VMEM((2, _RCHUNK, _NP), jnp.float32),
        pltpu.VMEM((2, _RCHUNK, _N_UNMASKED), jnp.float32),
        pltpu.VMEM((2 * _N_UNMASKED,), jnp.int32),
        pltpu.SemaphoreType.DMA,
        pltpu.SemaphoreType.DMA,
    ],
)
def _sc_compact(emb_hbm, idx_hbm, out_hbm, in_v, out_v, idx_v, in_sem, out_sem):
    wid = lax.axis_index("s") * 2 + lax.axis_index("c")
    row0 = wid * _ROWS_PER_TILE
    b0 = (wid * 5) // 4
    boundary = (b0 + 1) * _D
    pltpu.sync_copy(idx_hbm.at[pl.ds(b0 * _N_UNMASKED, 2 * _N_UNMASKED)],
                    idx_v)

    def start_in(gkk, buf):
        pltpu.async_copy(emb_hbm.at[pl.ds(row0 + gkk * _RCHUNK, _RCHUNK)],
                         in_v.at[buf], in_sem)

    def wait_in(buf):
        pltpu.make_async_copy(emb_hbm.at[pl.ds(0, _RCHUNK)], in_v.at[buf],
                              in_sem).wait()

    def start_out(gkk, buf):
        pltpu.async_copy(out_v.at[buf],
                         out_hbm.at[pl.ds(row0 + gkk * _RCHUNK, _RCHUNK)],
                         out_sem)

    def wait_out(buf):
        pltpu.make_async_copy(out_v.at[buf], out_hbm.at[pl.ds(0, _RCHUNK)],
                              out_sem).wait()

    start_in(0, 0)

    def pair_body(p, c):
        for parity in (0, 1):
            gkk = 2 * p + parity

            @pl.when(gkk + 1 < 2 * (_NCHUNK // 2))
            def _():
                start_in(gkk + 1, 1 - parity)

            wait_in(parity)
            @pl.when(gkk >= 2)
            def _():
                wait_out(parity)
            ioff = jnp.where(row0 + gkk * _RCHUNK >= boundary, _N_UNMASKED, 0)
            idxs = tuple(
                idx_v[pl.ds(ioff + j * _LANES, _LANES)]
                for j in range(_N_UNMASKED // _LANES))

            def row_body(r, rvec):
                for j in range(_N_UNMASKED // _LANES):
                    out_v[parity, r, pl.ds(j * _LANES, _LANES)] = \
                        plsc.load_gather(in_v.at[parity], [rvec, idxs[j]])
                return rvec + 1

            lax.fori_loop(0, _RCHUNK, row_body,
                          jnp.zeros((_LANES,), jnp.int32), unroll=False)
            start_out(gkk, parity)
        return c

    lax.fori_loop(0, _NCHUNK // 2, pair_body, 0, unroll=False)
    wait_out(0)
    wait_out(1)


def _tc_body(emb_ref, idx_ref, out_ref):
    idxrow = idx_ref[0, 0, :]
    iota = lax.broadcasted_iota(jnp.int32, (_NP, _N_UNMASKED), 0)
    onehot = (iota == idxrow[None, :]).astype(jnp.float32)
    out_ref[0] = jax.lax.dot_general(
        emb_ref[0], onehot, (((1,), (0,)), ((), ())),
        preferred_element_type=jnp.float32,
        precision=jax.lax.Precision.HIGHEST)


_tc_select = pl.pallas_call(
    _tc_body,
    grid=(_TC_B,),
    in_specs=[
        pl.BlockSpec((1, _D, _NP), lambda i: (_SC_B + i, 0, 0)),
        pl.BlockSpec((1, 1, _N_UNMASKED), lambda i: (i, 0, 0)),
    ],
    out_specs=pl.BlockSpec((1, _D, _N_UNMASKED), lambda i: (i, 0, 0)),
    out_shape=jax.ShapeDtypeStruct((_TC_B, _D, _N_UNMASKED), jnp.float32),
)


def kernel(patch_embeddings):
    masked, unmasked, bool_mask = _CONSTS
    emb_t3 = jnp.transpose(patch_embeddings, (0, 2, 1))
    emb_t = emb_t3.reshape(_B * _D, _NP)
    sc_out = _sc_compact(emb_t, jnp.asarray(unmasked[:_SC_B].reshape(-1)))
    tc_out = _tc_select(emb_t3,
                        jnp.asarray(unmasked[_SC_B:].reshape(_TC_B, 1,
                                                             _N_UNMASKED)))
    out = jnp.concatenate(
        [sc_out.reshape(_SC_B, _D, _N_UNMASKED), tc_out], axis=0)
    unmasked_patches = jnp.transpose(out, (0, 2, 1))
    return (
        unmasked_patches,
        jnp.asarray(bool_mask),
        jnp.asarray(masked),
        jnp.asarray(unmasked),
    )
